# baseline (speedup 1.0000x reference)
import functools

import jax
import jax.numpy as jnp
from jax import lax
from jax.experimental import pallas as pl
from jax.experimental.pallas import tpu as pltpu
from jax.experimental.pallas import tpu_sc as plsc

_V = 128
_D = 49152
_B = 1024
_NC = 2
_NS = 16
_NW = _NC * _NS
_BPW = _B // _NW
_C = 1536
_NCHUNK = _D // _C
_L = 16


def _gather_body(table_hbm, val_hbm, pos_hbm, out_hbm,
                 vp_v, cache0, cache1,
                 gsem0, gsem1, ssem0, ssem1):
    cid = lax.axis_index("c")
    sid = lax.axis_index("s")
    wid = sid * _NC + cid
    base = wid * _BPW
    pltpu.sync_copy(val_hbm.at[pl.ds(base, _BPW)], vp_v.at[0])
    pltpu.sync_copy(pos_hbm.at[pl.ds(base, _BPW)], vp_v.at[1])

    vals, poss = [], []
    for v in range(_BPW // _L):
        vvec = vp_v[0, pl.ds(v * _L, _L)]
        pvec = vp_v[1, pl.ds(v * _L, _L)]
        for j in range(_L):
            vals.append(vvec[j])
            poss.append(pvec[j])

    fresh = [None] * _BPW
    slot = [None] * _BPW
    fresh[0] = jnp.bool_(True)
    slot[0] = jnp.int32(0)
    for j in range(1, _BPW):
        fresh[j] = vals[j] != vals[j - 1]
        slot[j] = slot[j - 1] + fresh[j].astype(jnp.int32)

    caches = (cache0, cache1)
    gsems = (gsem0, gsem1)
    ssems = (ssem0, ssem1)

    def gather_desc(j, c, buf):
        off = pl.multiple_of(c * _C, _C)
        return pltpu.make_async_copy(
            table_hbm.at[pl.ds(vals[j], 1), pl.ds(off, _C)],
            caches[buf].at[pl.ds(slot[j], 1)],
            gsems[buf])

    def store_desc(j, c, buf):
        off = pl.multiple_of(c * _C, _C)
        return pltpu.make_async_copy(
            caches[buf].at[pl.ds(slot[j], 1)],
            out_hbm.at[pl.ds(poss[j], 1), pl.ds(off, _C)],
            ssems[buf])

    def do_chunk(c, buf):
        @pl.when(c >= 2)
        def _():
            for j in range(_BPW):
                store_desc(j, c - 2, buf).wait()

        for j in range(_BPW):
            @pl.when(fresh[j])
            def _(j=j):
                gather_desc(j, c, buf).start()
        for j in range(_BPW):
            @pl.when(fresh[j])
            def _(j=j):
                gather_desc(j, c, buf).wait()
        for j in range(_BPW):
            store_desc(j, c, buf).start()

    def pair_body(p, carry):
        do_chunk(p * 2, 0)
        do_chunk(p * 2 + 1, 1)
        return carry

    lax.fori_loop(0, _NCHUNK // 2, pair_body, 0)

    for j in range(_BPW):
        store_desc(j, _NCHUNK - 2, 0).wait()
    for j in range(_BPW):
        store_desc(j, _NCHUNK - 1, 1).wait()


@jax.jit
def _gather(table, val, pos):
    mesh = plsc.VectorSubcoreMesh(core_axis_name="c", subcore_axis_name="s")
    f = pl.kernel(
        _gather_body,
        out_type=jax.ShapeDtypeStruct((_B, _D), jnp.float32),
        mesh=mesh,
        scratch_types=[
            pltpu.VMEM((2, _BPW), jnp.int32),
            pltpu.VMEM((_BPW, _C), jnp.float32),
            pltpu.VMEM((_BPW, _C), jnp.float32),
            pltpu.SemaphoreType.DMA,
            pltpu.SemaphoreType.DMA,
            pltpu.SemaphoreType.DMA,
            pltpu.SemaphoreType.DMA,
        ],
    )
    return f(table, val, pos)


def kernel(prefix, embedding):
    idx = prefix.reshape(-1).astype(jnp.int32)
    order = jnp.argsort(idx)
    val = idx[order]
    pos = order.astype(jnp.int32)
    out = _gather(embedding, val, pos)
    return out.reshape(prefix.shape[0], prefix.shape[1], _D)


# speedup vs baseline: 1.9813x; 1.9813x over previous
import functools

import jax
import jax.numpy as jnp
from jax import lax
from jax.experimental import pallas as pl
from jax.experimental.pallas import tpu as pltpu
from jax.experimental.pallas import tpu_sc as plsc

_V = 128
_D = 49152
_B = 1024
_NC = 2
_NS = 16
_NW = _NC * _NS
_BPW = _B // _NW
_C = 2048
_NCHUNK = _D // _C
_RPT = _V // _NS
_L = 16
_H = _BPW // 2


def _gather_body(table_hbm, idx_hbm, out_hbm,
                 idx_v, rbuf0, rbuf1, sbuf0, sbuf1,
                 lsem0, lsem1, gsem0, gsem1, ssem0, ssem1):
    cid = lax.axis_index("c")
    sid = lax.axis_index("s")
    wid = sid * _NC + cid
    base = wid * _BPW
    pltpu.sync_copy(idx_hbm.at[pl.ds(base, _BPW)], idx_v)

    scalars = []
    for v in range(_BPW // _L):
        vec = idx_v[pl.ds(v * _L, _L)]
        for j in range(_L):
            scalars.append(vec[j])

    rbufs = (rbuf0, rbuf1)
    sbufs = (sbuf0, sbuf1)
    lsems = (lsem0, lsem1)
    gsems = (gsem0, gsem1)
    ssems = (ssem0, ssem1)
    row0 = sid * _RPT

    def load_desc(c, slot):
        off = pl.multiple_of(c * _C, _C)
        return pltpu.make_async_copy(
            table_hbm.at[pl.ds(row0, _RPT), pl.ds(off, _C)],
            sbufs[slot].at[pl.ds(row0, _RPT)],
            lsems[slot])

    def store_desc(c, h):
        off = pl.multiple_of(c * _C, _C)
        return pltpu.make_async_copy(
            rbufs[h],
            out_hbm.at[pl.ds(base + h * _H, _H), pl.ds(off, _C)],
            ssems[h])

    def do_chunk(c, slot, first):
        load_desc(c, slot).wait()
        plsc.subcore_barrier()

        @pl.when(c + 1 < _NCHUNK)
        def _():
            load_desc(c + 1, 1 - slot).start()

        for h in range(2):
            if first:
                @pl.when(c >= 1)
                def _(h=h):
                    store_desc(c - 1, h).wait()
            else:
                store_desc(c - 1, h).wait()
            copies = []
            for j in range(_H):
                cp = pltpu.make_async_copy(
                    sbufs[slot].at[pl.ds(scalars[h * _H + j], 1)],
                    rbufs[h].at[pl.ds(j, 1)],
                    gsems[h])
                cp.start()
                copies.append(cp)
            for cp in copies:
                cp.wait()
            store_desc(c, h).start()

    load_desc(0, 0).start()

    def pair_body(p, carry):
        c = p * 2
        do_chunk(c, 0, True)
        do_chunk(c + 1, 1, False)
        return carry

    lax.fori_loop(0, _NCHUNK // 2, pair_body, 0)

    store_desc(_NCHUNK - 1, 0).wait()
    store_desc(_NCHUNK - 1, 1).wait()


@jax.jit
def _gather(table, idx):
    mesh = plsc.VectorSubcoreMesh(core_axis_name="c", subcore_axis_name="s")
    f = pl.kernel(
        _gather_body,
        out_type=jax.ShapeDtypeStruct((_B, _D), jnp.float32),
        mesh=mesh,
        scratch_types=[
            pltpu.VMEM((_BPW,), jnp.int32),
            pltpu.VMEM((_H, _C), jnp.float32),
            pltpu.VMEM((_H, _C), jnp.float32),
            pltpu.VMEM_SHARED((_V, _C), jnp.float32),
            pltpu.VMEM_SHARED((_V, _C), jnp.float32),
            pltpu.SemaphoreType.DMA,
            pltpu.SemaphoreType.DMA,
            pltpu.SemaphoreType.DMA,
            pltpu.SemaphoreType.DMA,
            pltpu.SemaphoreType.DMA,
            pltpu.SemaphoreType.DMA,
        ],
    )
    return f(table, idx)


def kernel(prefix, embedding):
    idx = prefix.reshape(-1).astype(jnp.int32)
    out = _gather(embedding, idx)
    return out.reshape(prefix.shape[0], prefix.shape[1], _D)


# speedup vs baseline: 1.9998x; 1.0093x over previous
import functools

import jax
import jax.numpy as jnp
from jax import lax
from jax.experimental import pallas as pl
from jax.experimental.pallas import tpu as pltpu
from jax.experimental.pallas import tpu_sc as plsc

_V = 128
_D = 49152
_B = 1024
_NC = 2
_NS = 16
_DPC = _D // _NC
_BPT = _B // _NS
_C = 1536
_NCHUNK = _DPC // _C
_RPT = _V // _NS
_L = 16
_H = _BPT // 2


def _gather_body(table_hbm, idx_hbm, out_hbm,
                 idx_v, rbuf0, rbuf1, sbuf0, sbuf1,
                 lsem0, lsem1, gsem0, gsem1, ssem0, ssem1):
    cid = lax.axis_index("c")
    sid = lax.axis_index("s")
    col0 = cid * _DPC
    base = sid * _BPT
    pltpu.sync_copy(idx_hbm.at[pl.ds(base, _BPT)], idx_v)

    scalars = []
    for v in range(_BPT // _L):
        vec = idx_v[pl.ds(v * _L, _L)]
        for j in range(_L):
            scalars.append(vec[j])

    rbufs = (rbuf0, rbuf1)
    sbufs = (sbuf0, sbuf1)
    lsems = (lsem0, lsem1)
    gsems = (gsem0, gsem1)
    ssems = (ssem0, ssem1)
    row0 = sid * _RPT

    def load_desc(c, slot):
        off = pl.multiple_of(col0 + c * _C, _C)
        return pltpu.make_async_copy(
            table_hbm.at[pl.ds(row0, _RPT), pl.ds(off, _C)],
            sbufs[slot].at[pl.ds(row0, _RPT)],
            lsems[slot])

    def store_desc(c, h):
        off = pl.multiple_of(col0 + c * _C, _C)
        return pltpu.make_async_copy(
            rbufs[h],
            out_hbm.at[pl.ds(base + h * _H, _H), pl.ds(off, _C)],
            ssems[h])

    def do_chunk(c, slot, first):
        load_desc(c, slot).wait()
        plsc.subcore_barrier()

        @pl.when(c + 1 < _NCHUNK)
        def _():
            load_desc(c + 1, 1 - slot).start()

        for h in range(2):
            if first:
                @pl.when(c >= 1)
                def _(h=h):
                    store_desc(c - 1, h).wait()
            else:
                store_desc(c - 1, h).wait()
            copies = []
            for j in range(_H):
                cp = pltpu.make_async_copy(
                    sbufs[slot].at[pl.ds(scalars[h * _H + j], 1)],
                    rbufs[h].at[pl.ds(j, 1)],
                    gsems[h])
                cp.start()
                copies.append(cp)
            for cp in copies:
                cp.wait()
            store_desc(c, h).start()

    load_desc(0, 0).start()

    def pair_body(p, carry):
        c = p * 2
        do_chunk(c, 0, True)
        do_chunk(c + 1, 1, False)
        return carry

    lax.fori_loop(0, _NCHUNK // 2, pair_body, 0)

    store_desc(_NCHUNK - 1, 0).wait()
    store_desc(_NCHUNK - 1, 1).wait()


@jax.jit
def _gather(table, idx):
    mesh = plsc.VectorSubcoreMesh(core_axis_name="c", subcore_axis_name="s")
    f = pl.kernel(
        _gather_body,
        out_type=jax.ShapeDtypeStruct((_B, _D), jnp.float32),
        mesh=mesh,
        scratch_types=[
            pltpu.VMEM((_BPT,), jnp.int32),
            pltpu.VMEM((_H, _C), jnp.float32),
            pltpu.VMEM((_H, _C), jnp.float32),
            pltpu.VMEM_SHARED((_V, _C), jnp.float32),
            pltpu.VMEM_SHARED((_V, _C), jnp.float32),
            pltpu.SemaphoreType.DMA,
            pltpu.SemaphoreType.DMA,
            pltpu.SemaphoreType.DMA,
            pltpu.SemaphoreType.DMA,
            pltpu.SemaphoreType.DMA,
            pltpu.SemaphoreType.DMA,
        ],
    )
    return f(table, idx)


def kernel(prefix, embedding):
    idx = prefix.reshape(-1).astype(jnp.int32)
    out = _gather(embedding, idx)
    return out.reshape(prefix.shape[0], prefix.shape[1], _D)
